# baseline (speedup 1.0000x reference)
import functools

import jax
import jax.numpy as jnp
from jax import lax
from jax.experimental import pallas as pl
from jax.experimental.pallas import tpu as pltpu
from jax.experimental.pallas import tpu_sc as plsc

_VOCAB = 1000
_TBL_PAD = 1024
_PAD = 1


def _table_body(emb_ref, w_ref, b_ref, out_ref):
    t = lax.dot_general(
        w_ref[...], emb_ref[...], (((0,), (1,)), ((), ())),
        preferred_element_type=jnp.float32,
    )
    col = lax.broadcasted_iota(jnp.int32, t.shape, 1)
    out_ref[...] = jnp.where(col == _PAD, 0.0, t) + b_ref[...]


def _build_table(emb, w, b2):
    return pl.pallas_call(
        _table_body,
        out_shape=jax.ShapeDtypeStruct((1, _VOCAB), jnp.float32),
    )(emb, w, b2)


def _sc_lookup(table_row, idx):
    info = plsc.get_sparse_core_info()
    nw = info.num_cores * info.num_subcores
    lanes = info.num_lanes
    batch = idx.shape[0]
    bpw = batch // nw
    mesh = plsc.VectorSubcoreMesh(core_axis_name="c", subcore_axis_name="s")

    @functools.partial(
        pl.kernel,
        out_type=jax.ShapeDtypeStruct((batch,), jnp.float32),
        mesh=mesh,
        scratch_types=[
            pltpu.VMEM((_TBL_PAD,), jnp.float32),
            pltpu.VMEM((bpw,), jnp.int32),
            pltpu.VMEM((bpw,), jnp.float32),
            pltpu.SemaphoreType.DMA,
            pltpu.SemaphoreType.DMA,
        ],
        compiler_params=pltpu.CompilerParams(
            needs_layout_passes=False, skip_device_barrier=True
        ),
    )
    def k(table_hbm, idx_hbm, out_hbm, table_v, idx_v, out_v, sem_t, sem_i):
        wid = lax.axis_index("s") * info.num_cores + lax.axis_index("c")
        base = wid * bpw
        tbl_cp = pltpu.async_copy(table_hbm.at[0], table_v.at[pl.ds(0, _VOCAB)], sem_t)
        idx_cp = pltpu.async_copy(idx_hbm.at[pl.ds(base, bpw)], idx_v, sem_i)
        tbl_cp.wait()
        idx_cp.wait()
        for j in range(bpw // lanes):
            iv = idx_v[pl.ds(j * lanes, lanes)]
            out_v[pl.ds(j * lanes, lanes)] = plsc.load_gather(table_v, [iv])
        pltpu.sync_copy(out_v, out_hbm.at[pl.ds(base, bpw)])

    return k(table_row, idx)


def kernel(indices, emb, W, b):
    idx0 = indices[0, :, 0].astype(jnp.int32)
    table = _build_table(emb, W, b.reshape(1, 1))
    return _sc_lookup(table, idx0)[:, None]
